# baseline (speedup 1.0000x reference)
import functools

import jax
import jax.numpy as jnp
from jax.experimental import pallas as pl
from jax.experimental.pallas import tpu as pltpu

HIDDEN = 1024
NUM_EXPERTS = 8
TOP_K = 2
MOE_FF = 512

BT = 256
BLK = 256
A = 2048 * TOP_K
NB_MAX = A // BLK + NUM_EXPERTS
NP_MAX = NB_MAX * BLK


def _router_kernel(x_ref, gw_ref, w_ref, idx_ref):
    xb = x_ref[...]
    logits = jnp.dot(xb, gw_ref[...].T, preferred_element_type=jnp.float32)
    m = jnp.max(logits, axis=1, keepdims=True)
    p = jnp.exp(logits - m)
    prob = p / jnp.sum(p, axis=1, keepdims=True)
    iota_e = jax.lax.broadcasted_iota(jnp.int32, prob.shape, 1)
    i1 = jnp.argmax(prob, axis=1).astype(jnp.int32)
    w1 = jnp.max(prob, axis=1)
    masked = jnp.where(iota_e == i1[:, None], -1.0, prob)
    i2 = jnp.argmax(masked, axis=1).astype(jnp.int32)
    w2 = jnp.max(masked, axis=1)
    s = w1 + w2
    w_ref[...] = jnp.stack([w1 / s, w2 / s], axis=1)
    idx_ref[...] = jnp.stack([i1, i2], axis=1)


def _router(x, gate_w):
    T, H = x.shape
    E = NUM_EXPERTS
    return pl.pallas_call(
        _router_kernel,
        grid=(T // BT,),
        in_specs=[
            pl.BlockSpec((BT, H), lambda t: (t, 0)),
            pl.BlockSpec((E, H), lambda t: (0, 0)),
        ],
        out_specs=[
            pl.BlockSpec((BT, TOP_K), lambda t: (t, 0)),
            pl.BlockSpec((BT, TOP_K), lambda t: (t, 0)),
        ],
        out_shape=[
            jax.ShapeDtypeStruct((T, TOP_K), jnp.float32),
            jax.ShapeDtypeStruct((T, TOP_K), jnp.int32),
        ],
    )(x, gate_w)


def _bookkeeping(w, idx):
    T = w.shape[0]
    e_flat = idx.reshape(-1)
    order = jnp.argsort(e_flat)
    e_sorted = e_flat[order]
    counts = jnp.zeros((NUM_EXPERTS,), jnp.int32).at[e_flat].add(1)
    starts = jnp.cumsum(counts) - counts
    pc = ((counts + BLK - 1) // BLK) * BLK
    pstarts = jnp.cumsum(pc) - pc
    ends = pstarts + pc
    rank = jnp.arange(A, dtype=jnp.int32) - starts[e_sorted]
    dest = (pstarts[e_sorted] + rank).astype(jnp.int32)
    rows = jnp.zeros((NP_MAX,), jnp.int32).at[dest].set(
        (order // TOP_K).astype(jnp.int32))
    wrow = jnp.zeros((NP_MAX,), jnp.float32).at[dest].set(w.reshape(-1)[order])
    pos = jnp.zeros((A,), jnp.int32).at[order].set(dest).reshape(T, TOP_K)
    total = jnp.sum(pc)
    nblocks = (total // BLK).astype(jnp.int32)
    bstart = jnp.minimum(jnp.arange(NB_MAX, dtype=jnp.int32) * BLK, total - 1)
    block_e = jnp.minimum(
        jnp.sum(bstart[:, None] >= ends[None, :], axis=1),
        NUM_EXPERTS - 1).astype(jnp.int32)
    return rows, wrow, pos, nblocks, block_e


def _gm_kernel(be_ref, nb_ref, xs_ref, wg_ref, wu_ref, wd_ref, wr_ref, y_ref):
    i = pl.program_id(0)

    @pl.when(i < nb_ref[0])
    def _():
        xb = xs_ref[...].astype(jnp.bfloat16)
        g = jnp.dot(xb, wg_ref[0].astype(jnp.bfloat16).T,
                    preferred_element_type=jnp.float32)
        u = jnp.dot(xb, wu_ref[0].astype(jnp.bfloat16).T,
                    preferred_element_type=jnp.float32)
        act = g * jax.nn.sigmoid(g) * u
        o = jnp.dot(act.astype(jnp.bfloat16), wd_ref[0].astype(jnp.bfloat16).T,
                    preferred_element_type=jnp.float32)
        y_ref[...] = o * wr_ref[0, 0][:, None]


def _grouped_mlp(xs, Wg, Wu, Wd, wrow, nblocks, block_e):
    H, F, E = HIDDEN, MOE_FF, NUM_EXPERTS
    wrow3 = wrow.reshape(NB_MAX, 1, BLK)
    grid_spec = pltpu.PrefetchScalarGridSpec(
        num_scalar_prefetch=2,
        grid=(NB_MAX,),
        in_specs=[
            pl.BlockSpec((BLK, H), lambda i, be, nb: (i, 0)),
            pl.BlockSpec((1, F, H), lambda i, be, nb: (be[i], 0, 0)),
            pl.BlockSpec((1, F, H), lambda i, be, nb: (be[i], 0, 0)),
            pl.BlockSpec((1, H, F), lambda i, be, nb: (be[i], 0, 0)),
            pl.BlockSpec((1, 1, BLK), lambda i, be, nb: (i, 0, 0)),
        ],
        out_specs=pl.BlockSpec((BLK, H), lambda i, be, nb: (i, 0)),
    )
    return pl.pallas_call(
        _gm_kernel,
        grid_spec=grid_spec,
        out_shape=jax.ShapeDtypeStruct((NP_MAX, H), jnp.float32),
    )(block_e, nblocks.reshape(1), xs, Wg, Wu, Wd, wrow3)


def _combine_kernel(y0_ref, y1_ref, out_ref):
    out_ref[...] = y0_ref[...] + y1_ref[...]


def _combine(y0, y1):
    T, H = y0.shape
    return pl.pallas_call(
        _combine_kernel,
        grid=(T // BT,),
        in_specs=[
            pl.BlockSpec((BT, H), lambda t: (t, 0)),
            pl.BlockSpec((BT, H), lambda t: (t, 0)),
        ],
        out_specs=pl.BlockSpec((BT, H), lambda t: (t, 0)),
        out_shape=jax.ShapeDtypeStruct((T, H), jnp.float32),
    )(y0, y1)


@jax.jit
def _moe(x, gate_w, Wg, Wu, Wd):
    w, idx = _router(x, gate_w)
    rows, wrow, pos, nblocks, block_e = _bookkeeping(w, idx)
    xs = x[rows]
    y = _grouped_mlp(xs, Wg, Wu, Wd, wrow, nblocks, block_e)
    y0 = y[pos[:, 0]]
    y1 = y[pos[:, 1]]
    return _combine(y0, y1)


def kernel(hidden_states, gate_w, Wg, Wu, Wd):
    b, s, h = hidden_states.shape
    x = hidden_states.reshape(-1, h)
    out = _moe(x, gate_w, Wg, Wu, Wd)
    return out.reshape(b, s, h)
